# Initial kernel scaffold; baseline (speedup 1.0000x reference)
import functools

import jax
import jax.numpy as jnp
from jax import lax
from jax.experimental import pallas as pl
from jax.experimental.pallas import tpu as pltpu
from jax.experimental.pallas import tpu_sc as plsc

NC = 2
NS = 16
NW = NC * NS
LANES = 16
DW = 16


def _chunking(edges_per_tile):
    for k in (128, 112, 104, 100, 96, 80, 64, 50, 40, 32, 25, 16, 10, 8):
        if edges_per_tile % k == 0:
            return k, edges_per_tile // k
    raise ValueError(f"no chunking for {edges_per_tile}")


def _zero_fill(buf):
    rows, width = buf.shape
    z = jnp.zeros((LANES,), jnp.float32)

    @pl.loop(0, rows)
    def _(r):
        @pl.loop(0, width, step=LANES)
        def _(col):
            buf[r, pl.ds(col, LANES)] = z


def _stripe_init_zero(scratch_buf, acc, stripe_base, rpt, k):
    _zero_fill(scratch_buf)
    nfull, rem = rpt // k, rpt % k
    for i in range(nfull):
        pltpu.sync_copy(scratch_buf, acc.at[pl.ds(stripe_base + i * k, k)])
    if rem:
        pltpu.sync_copy(scratch_buf.at[pl.ds(0, rem)],
                        acc.at[pl.ds(stripe_base + nfull * k, rem)])


def _make_deg_kernel(n, nchunks, k):
    rpt = n // NS
    mesh = plsc.VectorSubcoreMesh(core_axis_name="c", subcore_axis_name="s")

    @functools.partial(
        pl.kernel,
        out_type=jax.ShapeDtypeStruct((NC, n, DW), jnp.float32),
        mesh=mesh,
        scratch_types=[
            pltpu.VMEM((nchunks, k), jnp.int32),
            pltpu.VMEM((k, DW), jnp.float32),
            pltpu.VMEM_SHARED((n, DW), jnp.float32),
        ],
    )
    def deg_kernel(dst_hbm, out_hbm, dstv, ones, acc):
        c = lax.axis_index("c")
        s = lax.axis_index("s")
        wid = s * NC + c
        _stripe_init_zero(ones, acc, s * rpt, rpt, k)
        one = jnp.full((LANES,), 1.0, jnp.float32)

        @pl.loop(0, k)
        def _(r):
            ones[r, pl.ds(0, LANES)] = one

        pltpu.sync_copy(dst_hbm.at[wid], dstv)
        plsc.subcore_barrier()

        @pl.loop(0, nchunks)
        def _(j):
            pltpu.sync_copy(ones, acc.at[dstv.at[j]], add=True)

        plsc.subcore_barrier()
        pltpu.sync_copy(acc.at[pl.ds(s * rpt, rpt)],
                        out_hbm.at[c, pl.ds(s * rpt, rpt)])

    return deg_kernel


def _make_agg_kernel(n, f, nchunks, k):
    rpt = n // NS
    mesh = plsc.VectorSubcoreMesh(core_axis_name="c", subcore_axis_name="s")

    @functools.partial(
        pl.kernel,
        out_type=jax.ShapeDtypeStruct((NC, n, f), jnp.float32),
        mesh=mesh,
        scratch_types=[
            pltpu.VMEM((nchunks, k), jnp.int32),
            pltpu.VMEM((nchunks, k), jnp.int32),
            pltpu.VMEM((k, f), jnp.float32),
            pltpu.VMEM_SHARED((n, f), jnp.float32),
        ],
    )
    def agg_kernel(tbl_hbm, src_hbm, dst_hbm, out_hbm, srcv, dstv, rows, acc):
        c = lax.axis_index("c")
        s = lax.axis_index("s")
        wid = s * NC + c
        _stripe_init_zero(rows, acc, s * rpt, rpt, k)
        pltpu.sync_copy(src_hbm.at[wid], srcv)
        pltpu.sync_copy(dst_hbm.at[wid], dstv)
        plsc.subcore_barrier()

        @pl.loop(0, nchunks)
        def _(j):
            pltpu.sync_copy(tbl_hbm.at[srcv.at[j]], rows)
            pltpu.sync_copy(rows, acc.at[dstv.at[j]], add=True)

        plsc.subcore_barrier()
        pltpu.sync_copy(acc.at[pl.ds(s * rpt, rpt)],
                        out_hbm.at[c, pl.ds(s * rpt, rpt)])

    return agg_kernel


def _prep_body(degp_ref, x_ref, dis_ref, xs_ref):
    d = degp_ref[0, :, 0:1] + degp_ref[1, :, 0:1] + 1.0
    dis = lax.rsqrt(d)
    dis_ref[...] = dis
    xs_ref[...] = x_ref[...] * dis


def _mid_body(p_ref, xs_ref, dis_ref, w1_ref, b1_ref, w2_ref, hs_ref):
    dis = dis_ref[...]
    agg = dis * (p_ref[0] + p_ref[1] + xs_ref[...])
    h1 = jnp.dot(agg, w1_ref[...], preferred_element_type=jnp.float32,
                 precision=lax.Precision.HIGHEST)
    h1 = jnp.maximum(h1 + b1_ref[...], 0.0)
    h2 = jnp.dot(h1, w2_ref[...], preferred_element_type=jnp.float32,
                 precision=lax.Precision.HIGHEST)
    hs_ref[...] = dis * h2


def _final_body(q_ref, hs_ref, dis_ref, b2_ref, out_ref, *, c_out):
    r = dis_ref[...] * (q_ref[0] + q_ref[1] + hs_ref[...]) + b2_ref[...]
    out_ref[...] = r[:, :c_out]


def kernel(x, edge_index, W1, b1, W2, b2):
    n, f_in = x.shape
    e = edge_index.shape[1]
    h = W1.shape[1]
    c_out = W2.shape[1]
    f2 = 48

    assert e % NW == 0 and n % NS == 0
    ept = e // NW
    k, nchunks = _chunking(ept)

    src3 = edge_index[0].reshape(NW, nchunks, k)
    dst3 = edge_index[1].reshape(NW, nchunks, k)
    w2p = jnp.pad(W2, ((0, 0), (0, f2 - c_out)))
    b2p = jnp.pad(b2, (0, f2 - c_out)).reshape(1, f2)
    b1r = b1.reshape(1, h)

    degp = _make_deg_kernel(n, nchunks, k)(dst3)

    dis, xs = pl.pallas_call(
        _prep_body,
        out_shape=(jax.ShapeDtypeStruct((n, 1), jnp.float32),
                   jax.ShapeDtypeStruct((n, f_in), jnp.float32)),
    )(degp, x)

    p = _make_agg_kernel(n, f_in, nchunks, k)(xs, src3, dst3)

    nb = 2000
    grid = (n // nb,)
    hs = pl.pallas_call(
        _mid_body,
        grid=grid,
        in_specs=[
            pl.BlockSpec((NC, nb, f_in), lambda i: (0, i, 0)),
            pl.BlockSpec((nb, f_in), lambda i: (i, 0)),
            pl.BlockSpec((nb, 1), lambda i: (i, 0)),
            pl.BlockSpec((f_in, h), lambda i: (0, 0)),
            pl.BlockSpec((1, h), lambda i: (0, 0)),
            pl.BlockSpec((h, f2), lambda i: (0, 0)),
        ],
        out_specs=pl.BlockSpec((nb, f2), lambda i: (i, 0)),
        out_shape=jax.ShapeDtypeStruct((n, f2), jnp.float32),
    )(p, xs, dis, W1, b1r, w2p)

    q = _make_agg_kernel(n, f2, nchunks, k)(hs, src3, dst3)

    out = pl.pallas_call(
        functools.partial(_final_body, c_out=c_out),
        grid=grid,
        in_specs=[
            pl.BlockSpec((NC, nb, f2), lambda i: (0, i, 0)),
            pl.BlockSpec((nb, f2), lambda i: (i, 0)),
            pl.BlockSpec((nb, 1), lambda i: (i, 0)),
            pl.BlockSpec((1, f2), lambda i: (0, 0)),
        ],
        out_specs=pl.BlockSpec((nb, c_out), lambda i: (i, 0)),
        out_shape=jax.ShapeDtypeStruct((n, c_out), jnp.float32),
    )(q, hs, dis, b2p)

    return out


# speedup vs baseline: 20.6849x; 20.6849x over previous
import functools

import jax
import jax.numpy as jnp
from jax import lax
from jax.experimental import pallas as pl
from jax.experimental.pallas import tpu as pltpu
from jax.experimental.pallas import tpu_sc as plsc

NC = 2
NS = 16
NW = NC * NS
LANES = 16
DW = 128


def _chunking(edges_per_tile):
    for k in (128, 112, 104, 100, 96, 80, 64, 50, 40, 32, 25, 16, 10, 8):
        if edges_per_tile % k == 0:
            return k, edges_per_tile // k
    raise ValueError(f"no chunking for {edges_per_tile}")


def _zero_fill(buf):
    rows, width = buf.shape
    z = jnp.zeros((LANES,), jnp.float32)

    @pl.loop(0, rows)
    def _(r):
        @pl.loop(0, width, step=LANES)
        def _(col):
            buf[r, pl.ds(col, LANES)] = z


def _stripes(n):
    base = (n // NS) // 8 * 8
    rem = n - NS * base
    assert rem % 8 == 0
    return base, rem


def _fill_stripe(scratch_buf, acc, stripe_base, nrows, ic):
    nfull, rem = nrows // ic, nrows % ic
    for i in range(nfull):
        pltpu.sync_copy(scratch_buf.at[pl.ds(0, ic)],
                        acc.at[pl.ds(stripe_base + i * ic, ic)])
    if rem:
        pltpu.sync_copy(scratch_buf.at[pl.ds(0, rem)],
                        acc.at[pl.ds(stripe_base + nfull * ic, rem)])


def _init_and_drain(k):
    ic = min(k, 96) // 8 * 8

    def init(scratch_buf, acc, s, base, rem):
        _fill_stripe(scratch_buf, acc, s * base, base, ic)
        if rem:
            @pl.when(s == NS - 1)
            def _():
                _fill_stripe(scratch_buf, acc, NS * base, rem, ic)

    def drain(acc, out_hbm, c, s, base, rem):
        pltpu.sync_copy(acc.at[pl.ds(s * base, base)],
                        out_hbm.at[c, pl.ds(s * base, base)])
        if rem:
            @pl.when(s == NS - 1)
            def _():
                pltpu.sync_copy(acc.at[pl.ds(NS * base, rem)],
                                out_hbm.at[c, pl.ds(NS * base, rem)])

    return init, drain


def _make_deg_kernel(n, nchunks, k):
    base, rem = _stripes(n)
    init, drain = _init_and_drain(k)
    mesh = plsc.VectorSubcoreMesh(core_axis_name="c", subcore_axis_name="s")

    @functools.partial(
        pl.kernel,
        out_type=jax.ShapeDtypeStruct((NC, n, DW), jnp.float32),
        mesh=mesh,
        scratch_types=[
            pltpu.VMEM((nchunks, k), jnp.int32),
            pltpu.VMEM((k, DW), jnp.float32),
            pltpu.VMEM_SHARED((n, DW), jnp.float32),
        ],
    )
    def deg_kernel(dst_hbm, out_hbm, dstv, ones, acc):
        c = lax.axis_index("c")
        s = lax.axis_index("s")
        wid = s * NC + c
        _zero_fill(ones)
        init(ones, acc, s, base, rem)
        one = jnp.full((LANES,), 1.0, jnp.float32)

        @pl.loop(0, k)
        def _(r):
            ones[r, pl.ds(0, LANES)] = one

        pltpu.sync_copy(dst_hbm.at[wid], dstv)
        plsc.subcore_barrier()

        @pl.loop(0, nchunks)
        def _(j):
            pltpu.sync_copy(ones, acc.at[dstv.at[j]], add=True)

        plsc.subcore_barrier()
        drain(acc, out_hbm, c, s, base, rem)

    return deg_kernel


def _make_agg_kernel(n, f, nchunks, k):
    base, rem = _stripes(n)
    init, drain = _init_and_drain(k)
    mesh = plsc.VectorSubcoreMesh(core_axis_name="c", subcore_axis_name="s")

    @functools.partial(
        pl.kernel,
        out_type=jax.ShapeDtypeStruct((NC, n, f), jnp.float32),
        mesh=mesh,
        scratch_types=[
            pltpu.VMEM((nchunks, k), jnp.int32),
            pltpu.VMEM((nchunks, k), jnp.int32),
            pltpu.VMEM((k, f), jnp.float32),
            pltpu.VMEM_SHARED((n, f), jnp.float32),
        ],
    )
    def agg_kernel(tbl_hbm, src_hbm, dst_hbm, out_hbm, srcv, dstv, rows, acc):
        c = lax.axis_index("c")
        s = lax.axis_index("s")
        wid = s * NC + c
        _zero_fill(rows)
        init(rows, acc, s, base, rem)
        pltpu.sync_copy(src_hbm.at[wid], srcv)
        pltpu.sync_copy(dst_hbm.at[wid], dstv)
        plsc.subcore_barrier()

        @pl.loop(0, nchunks)
        def _(j):
            pltpu.sync_copy(tbl_hbm.at[srcv.at[j]], rows)
            pltpu.sync_copy(rows, acc.at[dstv.at[j]], add=True)

        plsc.subcore_barrier()
        drain(acc, out_hbm, c, s, base, rem)

    return agg_kernel


def _prep_body(degp_ref, x_ref, dis_ref, xs_ref):
    d = degp_ref[0, :, 0:1] + degp_ref[1, :, 0:1] + 1.0
    dis = lax.rsqrt(d)
    dis_ref[...] = dis
    xs_ref[...] = x_ref[...] * dis


def _mid_body(p_ref, xs_ref, dis_ref, w1_ref, b1_ref, w2_ref, hs_ref):
    dis = dis_ref[...]
    agg = dis * (p_ref[0] + p_ref[1] + xs_ref[...])
    h1 = jnp.dot(agg, w1_ref[...], preferred_element_type=jnp.float32,
                 precision=lax.Precision.HIGHEST)
    h1 = jnp.maximum(h1 + b1_ref[...], 0.0)
    h2 = jnp.dot(h1, w2_ref[...], preferred_element_type=jnp.float32,
                 precision=lax.Precision.HIGHEST)
    hs_ref[...] = dis * h2


def _final_body(q_ref, hs_ref, dis_ref, b2_ref, out_ref, *, c_out):
    r = dis_ref[...] * (q_ref[0] + q_ref[1] + hs_ref[...]) + b2_ref[...]
    out_ref[...] = r[:, :c_out]


def kernel(x, edge_index, W1, b1, W2, b2):
    n, f_in = x.shape
    e = edge_index.shape[1]
    h = W1.shape[1]
    c_out = W2.shape[1]
    f2 = 128

    assert e % NW == 0 and n % 8 == 0
    ept = e // NW
    k, nchunks = _chunking(ept)

    src3 = edge_index[0].reshape(NW, nchunks, k)
    dst3 = edge_index[1].reshape(NW, nchunks, k)
    w2p = jnp.pad(W2, ((0, 0), (0, f2 - c_out)))
    b2p = jnp.pad(b2, (0, f2 - c_out)).reshape(1, f2)
    b1r = b1.reshape(1, h)

    degp = _make_deg_kernel(n, nchunks, k)(dst3)

    dis, xs = pl.pallas_call(
        _prep_body,
        out_shape=(jax.ShapeDtypeStruct((n, 1), jnp.float32),
                   jax.ShapeDtypeStruct((n, f_in), jnp.float32)),
    )(degp, x)

    p = _make_agg_kernel(n, f_in, nchunks, k)(xs, src3, dst3)

    nb = 2000
    grid = (n // nb,)
    hs = pl.pallas_call(
        _mid_body,
        grid=grid,
        in_specs=[
            pl.BlockSpec((NC, nb, f_in), lambda i: (0, i, 0)),
            pl.BlockSpec((nb, f_in), lambda i: (i, 0)),
            pl.BlockSpec((nb, 1), lambda i: (i, 0)),
            pl.BlockSpec((f_in, h), lambda i: (0, 0)),
            pl.BlockSpec((1, h), lambda i: (0, 0)),
            pl.BlockSpec((h, f2), lambda i: (0, 0)),
        ],
        out_specs=pl.BlockSpec((nb, f2), lambda i: (i, 0)),
        out_shape=jax.ShapeDtypeStruct((n, f2), jnp.float32),
    )(p, xs, dis, W1, b1r, w2p)

    q = _make_agg_kernel(n, f2, nchunks, k)(hs, src3, dst3)

    out = pl.pallas_call(
        functools.partial(_final_body, c_out=c_out),
        grid=grid,
        in_specs=[
            pl.BlockSpec((NC, nb, f2), lambda i: (0, i, 0)),
            pl.BlockSpec((nb, f2), lambda i: (i, 0)),
            pl.BlockSpec((nb, 1), lambda i: (i, 0)),
            pl.BlockSpec((1, f2), lambda i: (0, 0)),
        ],
        out_specs=pl.BlockSpec((nb, c_out), lambda i: (i, 0)),
        out_shape=jax.ShapeDtypeStruct((n, c_out), jnp.float32),
    )(q, hs, dis, b2p)

    return out


# speedup vs baseline: 28.2024x; 1.3634x over previous
import functools

import jax
import jax.numpy as jnp
from jax import lax
from jax.experimental import pallas as pl
from jax.experimental.pallas import tpu as pltpu
from jax.experimental.pallas import tpu_sc as plsc

NC = 2
NS = 16
NW = NC * NS
LANES = 16
DW = 128


def _chunking(edges_per_tile, n):
    for k in (96, 80, 64, 48, 40, 32, 24, 16, 8):
        if edges_per_tile % k:
            continue
        nchunks = edges_per_tile // k
        words = NS * (edges_per_tile + nchunks * 128 + 2 * k * 128 + 2048)
        if words + 128 * n > 2_080_000:
            continue
        return k, nchunks
    raise ValueError(f"no chunking for {edges_per_tile}")


def _zero_fill(buf):
    rows, width = buf.shape
    z = jnp.zeros((LANES,), jnp.float32)

    @pl.loop(0, rows)
    def _(r):
        @pl.loop(0, width, step=LANES)
        def _(col):
            buf[r, pl.ds(col, LANES)] = z


def _stripes(n):
    base = (n // NS) // 8 * 8
    rem = n - NS * base
    assert rem % 8 == 0
    return base, rem


def _fill_stripe(scratch_buf, acc, stripe_base, nrows, ic):
    nfull, rem = nrows // ic, nrows % ic
    for i in range(nfull):
        pltpu.sync_copy(scratch_buf.at[pl.ds(0, ic)],
                        acc.at[pl.ds(stripe_base + i * ic, ic)])
    if rem:
        pltpu.sync_copy(scratch_buf.at[pl.ds(0, rem)],
                        acc.at[pl.ds(stripe_base + nfull * ic, rem)])


def _init_and_drain(k):
    ic = min(k, 96) // 8 * 8

    def init(scratch_buf, acc, s, base, rem):
        _fill_stripe(scratch_buf, acc, s * base, base, ic)
        if rem:
            @pl.when(s == NS - 1)
            def _():
                _fill_stripe(scratch_buf, acc, NS * base, rem, ic)

    def drain(acc, out_hbm, c, s, base, rem):
        pltpu.sync_copy(acc.at[pl.ds(s * base, base)],
                        out_hbm.at[c, pl.ds(s * base, base)])
        if rem:
            @pl.when(s == NS - 1)
            def _():
                pltpu.sync_copy(acc.at[pl.ds(NS * base, rem)],
                                out_hbm.at[c, pl.ds(NS * base, rem)])

    return init, drain


def _make_deg_kernel(n, nchunks, k):
    base, rem = _stripes(n)
    init, drain = _init_and_drain(k)
    mesh = plsc.VectorSubcoreMesh(core_axis_name="c", subcore_axis_name="s")

    @functools.partial(
        pl.kernel,
        out_type=jax.ShapeDtypeStruct((NC, n, DW), jnp.float32),
        mesh=mesh,
        scratch_types=[
            pltpu.VMEM((nchunks, k), jnp.int32),
            pltpu.VMEM((k, DW), jnp.float32),
            pltpu.VMEM_SHARED((n, DW), jnp.float32),
            pltpu.SemaphoreType.DMA,
        ],
    )
    def deg_kernel(dst_hbm, out_hbm, dstv, ones, acc, dsem):
        c = lax.axis_index("c")
        s = lax.axis_index("s")
        wid = s * NC + c
        _zero_fill(ones)
        init(ones, acc, s, base, rem)
        one = jnp.full((LANES,), 1.0, jnp.float32)

        @pl.loop(0, k)
        def _(r):
            ones[r, pl.ds(0, LANES)] = one

        pltpu.sync_copy(dst_hbm.at[wid], dstv)
        plsc.subcore_barrier()

        nloop = nchunks // 2 * 2
        pltpu.async_copy(ones, acc.at[dstv.at[0]], dsem, add=True)

        @pl.loop(0, nloop, step=2)
        def _(j):
            pltpu.async_copy(ones, acc.at[dstv.at[j + 1]], dsem, add=True)
            pltpu.make_async_copy(ones, acc.at[dstv.at[j]], dsem).wait()

            @pl.when(j + 2 < nchunks)
            def _():
                pltpu.async_copy(ones, acc.at[dstv.at[j + 2]], dsem, add=True)

            pltpu.make_async_copy(ones, acc.at[dstv.at[j + 1]], dsem).wait()

        if nchunks % 2:
            pltpu.make_async_copy(ones, acc.at[dstv.at[nchunks - 1]],
                                  dsem).wait()

        plsc.subcore_barrier()
        drain(acc, out_hbm, c, s, base, rem)

    return deg_kernel


def _make_agg_kernel(n, f, nchunks, k):
    base, rem = _stripes(n)
    init, drain = _init_and_drain(k)
    mesh = plsc.VectorSubcoreMesh(core_axis_name="c", subcore_axis_name="s")

    @functools.partial(
        pl.kernel,
        out_type=jax.ShapeDtypeStruct((NC, n, f), jnp.float32),
        mesh=mesh,
        scratch_types=[
            pltpu.VMEM((nchunks * k,), jnp.int32),
            pltpu.VMEM((nchunks, k), jnp.int32),
            pltpu.VMEM((2, k, f), jnp.float32),
            pltpu.VMEM_SHARED((n, f), jnp.float32),
            pltpu.SemaphoreType.DMA,
            pltpu.SemaphoreType.DMA,
        ],
    )
    def agg_kernel(tbl_hbm, src_hbm, dst_hbm, out_hbm, srcv, dstv, rows, acc,
                   sem0, sem1):
        c = lax.axis_index("c")
        s = lax.axis_index("s")
        wid = s * NC + c
        _zero_fill(rows.at[0])
        init(rows.at[0], acc, s, base, rem)
        pltpu.sync_copy(src_hbm.at[wid], srcv)
        pltpu.sync_copy(dst_hbm.at[wid], dstv)
        plsc.subcore_barrier()

        def gather(j, buf, sem):
            pltpu.async_copy(tbl_hbm.at[srcv.at[pl.ds(j * k, k)]],
                             rows.at[buf], sem)

        def gather_wait(buf, sem):
            pltpu.make_async_copy(tbl_hbm.at[srcv.at[pl.ds(0, k)]],
                                  rows.at[buf], sem).wait()

        def scatter(j, buf):
            pltpu.sync_copy(rows.at[buf], acc.at[dstv.at[j]], add=True)

        nloop = nchunks // 2 * 2
        gather(0, 0, sem0)

        @pl.loop(0, nloop, step=2)
        def _(j):
            gather(j + 1, 1, sem1)
            gather_wait(0, sem0)
            scatter(j, 0)

            @pl.when(j + 2 < nchunks)
            def _():
                gather(j + 2, 0, sem0)

            gather_wait(1, sem1)
            scatter(j + 1, 1)

        if nchunks % 2:
            gather_wait(0, sem0)
            scatter(nchunks - 1, 0)

        plsc.subcore_barrier()
        drain(acc, out_hbm, c, s, base, rem)

    return agg_kernel


def _prep_body(degp_ref, x_ref, dis_ref, xs_ref):
    d = degp_ref[0, :, 0:1] + degp_ref[1, :, 0:1] + 1.0
    dis = lax.rsqrt(d)
    dis_ref[...] = dis
    xs_ref[...] = x_ref[...] * dis


def _mid_body(p_ref, xs_ref, dis_ref, w1_ref, b1_ref, w2_ref, hs_ref):
    dis = dis_ref[...]
    agg = dis * (p_ref[0] + p_ref[1] + xs_ref[...])
    h1 = jnp.dot(agg, w1_ref[...], preferred_element_type=jnp.float32,
                 precision=lax.Precision.HIGHEST)
    h1 = jnp.maximum(h1 + b1_ref[...], 0.0)
    h2 = jnp.dot(h1, w2_ref[...], preferred_element_type=jnp.float32,
                 precision=lax.Precision.HIGHEST)
    hs_ref[...] = dis * h2


def _final_body(q_ref, hs_ref, dis_ref, b2_ref, out_ref, *, c_out):
    r = dis_ref[...] * (q_ref[0] + q_ref[1] + hs_ref[...]) + b2_ref[...]
    out_ref[...] = r[:, :c_out]


def kernel(x, edge_index, W1, b1, W2, b2):
    n, f_in = x.shape
    e = edge_index.shape[1]
    h = W1.shape[1]
    c_out = W2.shape[1]
    f2 = 128

    assert e % NW == 0 and n % 8 == 0
    ept = e // NW
    k, nchunks = _chunking(ept, n)

    src2 = edge_index[0].reshape(NW, nchunks * k)
    dst3 = edge_index[1].reshape(NW, nchunks, k)
    w2p = jnp.pad(W2, ((0, 0), (0, f2 - c_out)))
    b2p = jnp.pad(b2, (0, f2 - c_out)).reshape(1, f2)
    b1r = b1.reshape(1, h)

    degp = _make_deg_kernel(n, nchunks, k)(dst3)

    dis, xs = pl.pallas_call(
        _prep_body,
        out_shape=(jax.ShapeDtypeStruct((n, 1), jnp.float32),
                   jax.ShapeDtypeStruct((n, f_in), jnp.float32)),
    )(degp, x)

    p = _make_agg_kernel(n, f_in, nchunks, k)(xs, src2, dst3)

    nb = 2000
    grid = (n // nb,)
    hs = pl.pallas_call(
        _mid_body,
        grid=grid,
        in_specs=[
            pl.BlockSpec((NC, nb, f_in), lambda i: (0, i, 0)),
            pl.BlockSpec((nb, f_in), lambda i: (i, 0)),
            pl.BlockSpec((nb, 1), lambda i: (i, 0)),
            pl.BlockSpec((f_in, h), lambda i: (0, 0)),
            pl.BlockSpec((1, h), lambda i: (0, 0)),
            pl.BlockSpec((h, f2), lambda i: (0, 0)),
        ],
        out_specs=pl.BlockSpec((nb, f2), lambda i: (i, 0)),
        out_shape=jax.ShapeDtypeStruct((n, f2), jnp.float32),
    )(p, xs, dis, W1, b1r, w2p)

    q = _make_agg_kernel(n, f2, nchunks, k)(hs, src2, dst3)

    out = pl.pallas_call(
        functools.partial(_final_body, c_out=c_out),
        grid=grid,
        in_specs=[
            pl.BlockSpec((NC, nb, f2), lambda i: (0, i, 0)),
            pl.BlockSpec((nb, f2), lambda i: (i, 0)),
            pl.BlockSpec((nb, 1), lambda i: (i, 0)),
            pl.BlockSpec((1, f2), lambda i: (0, 0)),
        ],
        out_specs=pl.BlockSpec((nb, c_out), lambda i: (i, 0)),
        out_shape=jax.ShapeDtypeStruct((n, c_out), jnp.float32),
    )(q, hs, dis, b2p)

    return out


# speedup vs baseline: 34.6301x; 1.2279x over previous
import functools

import jax
import jax.numpy as jnp
from jax import lax
from jax.experimental import pallas as pl
from jax.experimental.pallas import tpu as pltpu
from jax.experimental.pallas import tpu_sc as plsc

NC = 2
NS = 16
NW = NC * NS
LANES = 16


def _chunking(edges_per_tile, n):
    for k in (96, 80, 64, 48, 40, 32, 24, 16, 8):
        if edges_per_tile % k:
            continue
        nchunks = edges_per_tile // k
        words = NS * (edges_per_tile + nchunks * 128 + 2 * k * 128 + 2048)
        if words + 128 * n > 2_080_000:
            continue
        return k, nchunks
    raise ValueError(f"no chunking for {edges_per_tile}")


def _zero_fill(buf):
    rows, width = buf.shape
    z = jnp.zeros((LANES,), jnp.float32)

    @pl.loop(0, rows)
    def _(r):
        @pl.loop(0, width, step=LANES)
        def _(col):
            buf[r, pl.ds(col, LANES)] = z


def _stripes(n):
    base = (n // NS) // 8 * 8
    rem = n - NS * base
    assert rem % 8 == 0
    return base, rem


def _fill_stripe(scratch_buf, acc, stripe_base, nrows, ic):
    nfull, rem = nrows // ic, nrows % ic
    for i in range(nfull):
        pltpu.sync_copy(scratch_buf.at[pl.ds(0, ic)],
                        acc.at[pl.ds(stripe_base + i * ic, ic)])
    if rem:
        pltpu.sync_copy(scratch_buf.at[pl.ds(0, rem)],
                        acc.at[pl.ds(stripe_base + nfull * ic, rem)])


def _init_and_drain(k):
    ic = min(k, 96) // 8 * 8

    def init(scratch_buf, acc, s, base, rem):
        _fill_stripe(scratch_buf, acc, s * base, base, ic)
        if rem:
            @pl.when(s == NS - 1)
            def _():
                _fill_stripe(scratch_buf, acc, NS * base, rem, ic)

    def drain(acc, out_hbm, c, s, base, rem):
        pltpu.sync_copy(acc.at[pl.ds(s * base, base)],
                        out_hbm.at[c, pl.ds(s * base, base)])
        if rem:
            @pl.when(s == NS - 1)
            def _():
                pltpu.sync_copy(acc.at[pl.ds(NS * base, rem)],
                                out_hbm.at[c, pl.ds(NS * base, rem)])

    return init, drain


def _make_deg_kernel(n, nchunks, k):
    base, rem = _stripes(n)
    bufn = -(-max(base, k) // LANES) * LANES
    mesh = plsc.VectorSubcoreMesh(core_axis_name="c", subcore_axis_name="s")

    @functools.partial(
        pl.kernel,
        out_type=jax.ShapeDtypeStruct((NC, n), jnp.float32),
        mesh=mesh,
        scratch_types=[
            pltpu.VMEM((nchunks, k), jnp.int32),
            pltpu.VMEM((bufn,), jnp.float32),
            pltpu.VMEM_SHARED((n,), jnp.float32),
            pltpu.SemaphoreType.DMA,
        ],
    )
    def deg_kernel(dst_hbm, out_hbm, dstv, ones, acc, dsem):
        c = lax.axis_index("c")
        s = lax.axis_index("s")
        wid = s * NC + c
        z = jnp.zeros((LANES,), jnp.float32)

        @pl.loop(0, bufn, step=LANES)
        def _(col):
            ones[pl.ds(col, LANES)] = z

        pltpu.sync_copy(ones.at[pl.ds(0, base)], acc.at[pl.ds(s * base, base)])
        if rem:
            @pl.when(s == NS - 1)
            def _():
                pltpu.sync_copy(ones.at[pl.ds(0, rem)],
                                acc.at[pl.ds(NS * base, rem)])

        one = jnp.full((LANES,), 1.0, jnp.float32)

        @pl.loop(0, k, step=LANES)
        def _(col):
            ones[pl.ds(col, LANES)] = one

        ones_k = ones.at[pl.ds(0, k)]
        pltpu.sync_copy(dst_hbm.at[wid], dstv)
        plsc.subcore_barrier()

        nloop = nchunks // 2 * 2
        pltpu.async_copy(ones_k, acc.at[dstv.at[0]], dsem, add=True)

        @pl.loop(0, nloop, step=2)
        def _(j):
            pltpu.async_copy(ones_k, acc.at[dstv.at[j + 1]], dsem, add=True)
            pltpu.make_async_copy(ones_k, acc.at[dstv.at[j]], dsem).wait()

            @pl.when(j + 2 < nchunks)
            def _():
                pltpu.async_copy(ones_k, acc.at[dstv.at[j + 2]], dsem,
                                 add=True)

            pltpu.make_async_copy(ones_k, acc.at[dstv.at[j + 1]], dsem).wait()

        if nchunks % 2:
            pltpu.make_async_copy(ones_k, acc.at[dstv.at[nchunks - 1]],
                                  dsem).wait()

        plsc.subcore_barrier()

        @pl.when(s == 0)
        def _():
            pltpu.sync_copy(acc, out_hbm.at[c])

    return deg_kernel


def _make_agg_kernel(n, f, nchunks, k):
    base, rem = _stripes(n)
    init, drain = _init_and_drain(k)
    mesh = plsc.VectorSubcoreMesh(core_axis_name="c", subcore_axis_name="s")

    @functools.partial(
        pl.kernel,
        out_type=jax.ShapeDtypeStruct((NC, n, f), jnp.float32),
        mesh=mesh,
        scratch_types=[
            pltpu.VMEM((nchunks * k,), jnp.int32),
            pltpu.VMEM((nchunks, k), jnp.int32),
            pltpu.VMEM((2, k, f), jnp.float32),
            pltpu.VMEM_SHARED((n, f), jnp.float32),
            pltpu.SemaphoreType.DMA,
            pltpu.SemaphoreType.DMA,
        ],
    )
    def agg_kernel(tbl_hbm, src_hbm, dst_hbm, out_hbm, srcv, dstv, rows, acc,
                   sem0, sem1):
        c = lax.axis_index("c")
        s = lax.axis_index("s")
        wid = s * NC + c
        _zero_fill(rows.at[0])
        init(rows.at[0], acc, s, base, rem)
        pltpu.sync_copy(src_hbm.at[wid], srcv)
        pltpu.sync_copy(dst_hbm.at[wid], dstv)
        plsc.subcore_barrier()

        def gather(j, buf, sem):
            pltpu.async_copy(tbl_hbm.at[srcv.at[pl.ds(j * k, k)]],
                             rows.at[buf], sem)

        def gather_wait(buf, sem):
            pltpu.make_async_copy(tbl_hbm.at[srcv.at[pl.ds(0, k)]],
                                  rows.at[buf], sem).wait()

        def scatter(j, buf):
            pltpu.sync_copy(rows.at[buf], acc.at[dstv.at[j]], add=True)

        nloop = nchunks // 2 * 2
        gather(0, 0, sem0)

        @pl.loop(0, nloop, step=2)
        def _(j):
            gather(j + 1, 1, sem1)
            gather_wait(0, sem0)
            scatter(j, 0)

            @pl.when(j + 2 < nchunks)
            def _():
                gather(j + 2, 0, sem0)

            gather_wait(1, sem1)
            scatter(j + 1, 1)

        if nchunks % 2:
            gather_wait(0, sem0)
            scatter(nchunks - 1, 0)

        plsc.subcore_barrier()
        drain(acc, out_hbm, c, s, base, rem)

    return agg_kernel


def _prep_body(degp_ref, x_ref, dis_ref, xs_ref):
    d = degp_ref[0, :, 0:1] + degp_ref[1, :, 0:1] + 1.0
    dis = lax.rsqrt(d)
    dis_ref[...] = dis
    xs_ref[...] = x_ref[...] * dis


def _mid_body(p_ref, xs_ref, dis_ref, w1_ref, b1_ref, w2_ref, hs_ref):
    dis = dis_ref[...]
    agg = dis * (p_ref[0] + p_ref[1] + xs_ref[...])
    h1 = jnp.dot(agg, w1_ref[...], preferred_element_type=jnp.float32,
                 precision=lax.Precision.DEFAULT)
    h1 = jnp.maximum(h1 + b1_ref[...], 0.0)
    h2 = jnp.dot(h1, w2_ref[...], preferred_element_type=jnp.float32,
                 precision=lax.Precision.DEFAULT)
    hs_ref[...] = dis * h2


def _final_body(q_ref, hs_ref, dis_ref, b2_ref, out_ref, *, c_out):
    r = dis_ref[...] * (q_ref[0] + q_ref[1] + hs_ref[...]) + b2_ref[...]
    out_ref[...] = r[:, :c_out]


def kernel(x, edge_index, W1, b1, W2, b2):
    n, f_in = x.shape
    e = edge_index.shape[1]
    h = W1.shape[1]
    c_out = W2.shape[1]
    f2 = 128

    assert e % NW == 0 and n % 8 == 0
    ept = e // NW
    k, nchunks = _chunking(ept, n)

    src2 = edge_index[0].reshape(NW, nchunks * k)
    dst3 = edge_index[1].reshape(NW, nchunks, k)
    w2p = jnp.pad(W2, ((0, 0), (0, f2 - c_out)))
    b2p = jnp.pad(b2, (0, f2 - c_out)).reshape(1, f2)
    b1r = b1.reshape(1, h)

    degp = _make_deg_kernel(n, nchunks, k)(dst3)
    degp = degp.reshape(NC, n, 1)

    dis, xs = pl.pallas_call(
        _prep_body,
        out_shape=(jax.ShapeDtypeStruct((n, 1), jnp.float32),
                   jax.ShapeDtypeStruct((n, f_in), jnp.float32)),
    )(degp, x)

    p = _make_agg_kernel(n, f_in, nchunks, k)(xs, src2, dst3)

    nb = 2000
    grid = (n // nb,)
    hs = pl.pallas_call(
        _mid_body,
        grid=grid,
        in_specs=[
            pl.BlockSpec((NC, nb, f_in), lambda i: (0, i, 0)),
            pl.BlockSpec((nb, f_in), lambda i: (i, 0)),
            pl.BlockSpec((nb, 1), lambda i: (i, 0)),
            pl.BlockSpec((f_in, h), lambda i: (0, 0)),
            pl.BlockSpec((1, h), lambda i: (0, 0)),
            pl.BlockSpec((h, f2), lambda i: (0, 0)),
        ],
        out_specs=pl.BlockSpec((nb, f2), lambda i: (i, 0)),
        out_shape=jax.ShapeDtypeStruct((n, f2), jnp.float32),
    )(p, xs, dis, W1, b1r, w2p)

    q = _make_agg_kernel(n, f2, nchunks, k)(hs, src2, dst3)

    out = pl.pallas_call(
        functools.partial(_final_body, c_out=c_out),
        grid=grid,
        in_specs=[
            pl.BlockSpec((NC, nb, f2), lambda i: (0, i, 0)),
            pl.BlockSpec((nb, f2), lambda i: (i, 0)),
            pl.BlockSpec((nb, 1), lambda i: (i, 0)),
            pl.BlockSpec((1, f2), lambda i: (0, 0)),
        ],
        out_specs=pl.BlockSpec((nb, c_out), lambda i: (i, 0)),
        out_shape=jax.ShapeDtypeStruct((n, c_out), jnp.float32),
    )(q, hs, dis, b2p)

    return out
